# baseline (speedup 1.0000x reference)
import functools

import numpy as np
import jax
import jax.numpy as jnp
from jax import lax
from jax.experimental import pallas as pl
from jax.experimental.pallas import tpu as pltpu
from jax.experimental.pallas import tpu_sc as plsc

_SEQ_LEN = 4096
_D_MODEL = 1024
_NUM_BLOCKS = 4
_CLS_ID = 2


def _rel_indices(seq_len: int, num_blocks: int) -> list[np.ndarray]:
    zero_offset = seq_len * 2
    pos = np.arange(seq_len)
    idx_list = []
    for b in range(num_blocks):
        if b > 0:
            cls_pos = np.array([-(2 ** b) + 1])
            pooled = np.concatenate([cls_pos, pos[1:-1][::2]])
            stride = 2 ** (b - 1)
            ref_point = pooled[0] - pos[0]
            num_remove = 2 * len(pooled)
            max_dist = ref_point + num_remove * stride
            min_dist = pooled[0] - pos[-1]
            idx_list.append(np.arange(max_dist, min_dist - 1, -stride) + zero_offset)
            pos = pooled
        stride = 2 ** b
        max_dist = len(pos) * stride
        min_dist = pos[0] - pos[-1]
        idx_list.append(np.arange(max_dist, min_dist - 1, -stride) + zero_offset)
    return idx_list


_SEGS = _rel_indices(_SEQ_LEN, _NUM_BLOCKS)
_NROWS = sum(len(s) for s in _SEGS)
_IDX_SC = np.concatenate(_SEGS).astype(np.int32)
_SC_ROWS = _IDX_SC.shape[0]

_NW = 32
_BPW = _SC_ROWS // _NW
_CH = 32
_NCH = _BPW // _CH


def _sc_gather(table, idx):
    mesh = plsc.VectorSubcoreMesh(core_axis_name="c", subcore_axis_name="s")

    @functools.partial(
        pl.kernel,
        mesh=mesh,
        out_type=jax.ShapeDtypeStruct((_SC_ROWS, _D_MODEL), jnp.float32),
        scratch_types=[
            pltpu.VMEM((_BPW,), jnp.int32),
            pltpu.VMEM((3, _CH, _D_MODEL), jnp.float32),
            pltpu.SemaphoreType.DMA,
            pltpu.SemaphoreType.DMA,
        ],
    )
    def k(table_hbm, idx_hbm, out_hbm, idx_v, buf_v, gsem, psem):
        wid = lax.axis_index("s") * 2 + lax.axis_index("c")
        base = pl.multiple_of(wid * _BPW, 8)
        pltpu.sync_copy(idx_hbm.at[pl.ds(base, _BPW)], idx_v)

        def gather(j):
            src = table_hbm.at[idx_v.at[pl.ds(j * _CH, _CH)]]
            return pltpu.async_copy(src, buf_v.at[j % 3], gsem)

        def put(j):
            dst = out_hbm.at[pl.ds(base + j * _CH, _CH)]
            return pltpu.async_copy(buf_v.at[j % 3], dst, psem)

        g = {0: gather(0), 1: gather(1)}
        p = {}
        waited = set()
        for j in range(_NCH):
            g[j].wait()
            p[j] = put(j)
            if j + 2 < _NCH:
                if j - 1 >= 0:
                    p[j - 1].wait()
                    waited.add(j - 1)
                g[j + 2] = gather(j + 2)
        for j in range(_NCH):
            if j not in waited:
                p[j].wait()

    return k(table, idx)


_BI = 512
_NI = _SEQ_LEN // _BI


def _mat_body(ids_row_ref, ids_col_ref, mat_ref):
    row = ids_row_ref[0]
    col = ids_col_ref[0]
    cls = jnp.int8(_CLS_ID)
    m = (col == row) | (col == cls) | (row == cls)
    mat_ref[0] = m.astype(jnp.int8)


def _cls_body(cls_ref):
    i = pl.program_id(0)
    r = lax.broadcasted_iota(jnp.int32, (_BI, _SEQ_LEN), 0) + i * _BI
    c = lax.broadcasted_iota(jnp.int32, (_BI, _SEQ_LEN), 1)
    cls_ref[...] = ((r > 0) & (c > 0)).astype(jnp.float32)


def _tc_mat(tti8):
    nb = tti8.shape[0]
    ids_row = tti8.reshape(nb, 1, _SEQ_LEN)
    ids_col = tti8.reshape(nb, _SEQ_LEN, 1)
    return pl.pallas_call(
        _mat_body,
        grid=(_NI, nb),
        in_specs=[
            pl.BlockSpec((1, 1, _SEQ_LEN), lambda i, b: (b, 0, 0)),
            pl.BlockSpec((1, _BI, 1), lambda i, b: (b, i, 0)),
        ],
        out_specs=pl.BlockSpec((1, _BI, _SEQ_LEN), lambda i, b: (b, i, 0)),
        out_shape=jax.ShapeDtypeStruct((nb, _SEQ_LEN, _SEQ_LEN), jnp.int8),
    )(ids_row, ids_col)


def _tc_cls():
    return pl.pallas_call(
        _cls_body,
        grid=(_NI,),
        out_specs=pl.BlockSpec((_BI, _SEQ_LEN), lambda i: (i, 0)),
        out_shape=jax.ShapeDtypeStruct((_SEQ_LEN, _SEQ_LEN), jnp.float32),
    )()


def kernel(pos_embed, token_type_ids):
    tti8 = token_type_ids.astype(jnp.int8)
    idx = jnp.asarray(_IDX_SC)
    pos_out = _sc_gather(pos_embed, idx)
    mat_i8 = _tc_mat(tti8)
    token_type_mat = mat_i8.view(jnp.bool_)
    cls_mask = _tc_cls()
    return (pos_out, token_type_mat, cls_mask)
